# baseline (speedup 1.0000x reference)
import dataclasses
import functools

import jax
import jax.numpy as jnp
from jax import lax
from jax.experimental import pallas as pl
from jax.experimental.pallas import tpu as pltpu
from jax.experimental.pallas import tpu_sc as plsc

ROWS = 128
COLS = 32768
K = 16
L = 16
NC = 2
NS = 16
G = 16
NG = COLS // (G * L)
U = 4


def _merge_max(a, b):
    return jnp.sort(jnp.maximum(a, jnp.flip(b)))


def _merge_min(a, b):
    return jnp.sort(jnp.minimum(a, jnp.flip(b)))


def _kv_merge_max(av, ai, bvd, bid):
    m = av >= bvd
    return plsc.sort_key_val(jnp.where(m, av, bvd), jnp.where(m, ai, bid))


def _kv_merge_min(av, ai, bvd, bid):
    m = av <= bvd
    return plsc.sort_key_val(jnp.where(m, av, bvd), jnp.where(m, ai, bid))


def kernel(x):
    nw = NC * NS
    rows_per_w = ROWS // nw

    mesh = plsc.VectorSubcoreMesh(core_axis_name="c", subcore_axis_name="s")

    cp = pltpu.CompilerParams()
    if "needs_layout_passes" in pltpu.CompilerParams.__dataclass_fields__:
        cp = dataclasses.replace(cp, needs_layout_passes=False)

    @functools.partial(
        pl.kernel,
        out_type=jax.ShapeDtypeStruct((ROWS, 2 * K), jnp.float32),
        mesh=mesh,
        compiler_params=cp,
        scratch_types=[
            pltpu.VMEM((COLS,), jnp.float32),
            pltpu.VMEM((COLS,), jnp.float32),
            pltpu.VMEM((2 * K,), jnp.float32),
            pltpu.VMEM((NG * L,), jnp.float32),
            pltpu.VMEM((NG * L,), jnp.float32),
            pltpu.SemaphoreType.DMA,
            pltpu.SemaphoreType.DMA,
        ],
    )
    def run(x_hbm, out_hbm, row_a, row_b, out_v, gmax_v, gmin_v,
            sem_a, sem_b):
        cid_ = lax.axis_index("c")
        sid_ = lax.axis_index("s")
        wid = sid_ * NC + cid_
        row0 = wid * rows_per_w

        neg = jnp.full((L,), -jnp.inf, jnp.float32)
        pos = jnp.full((L,), jnp.inf, jnp.float32)
        zero_ids = jnp.zeros((L,), jnp.int32)
        lane = lax.iota(jnp.int32, L)

        def compute_row(row, buf):
            def body_a(i, carry):
                for a in range(2):
                    g = i * 2 + a
                    gbase = g * (G * L)
                    c = buf[pl.ds(gbase, L)]
                    vmax = c
                    vmin = c
                    for k in range(1, G):
                        c = buf[pl.ds(gbase + k * L, L)]
                        vmax = jnp.maximum(vmax, c)
                        vmin = jnp.minimum(vmin, c)
                    gmax_v[pl.ds(g * L, L)] = vmax
                    gmin_v[pl.ds(g * L, L)] = vmin
                return carry

            lax.fori_loop(0, NG // 2, body_a, 0)

            def body_b(i, carry):
                kx = list(carry[0])
                ix = list(carry[1])
                kn = list(carry[2])
                im = list(carry[3])
                for a in range(U):
                    g = i * U + a
                    cid = jnp.broadcast_to(g * L, (L,)).astype(jnp.int32) + lane
                    vmax = gmax_v[pl.ds(g * L, L)]
                    svd, sid = plsc.sort_key_val(vmax, cid, descending=True)
                    kx[a], ix[a] = _kv_merge_max(kx[a], ix[a], svd, sid)
                    vmin = gmin_v[pl.ds(g * L, L)]
                    svd, sid = plsc.sort_key_val(vmin, cid, descending=True)
                    kn[a], im[a] = _kv_merge_min(kn[a], im[a], svd, sid)
                return tuple(kx), tuple(ix), tuple(kn), tuple(im)

            init = ((neg,) * U, (zero_ids,) * U, (pos,) * U, (zero_ids,) * U)
            kx, ix, kn, im = lax.fori_loop(0, NG // U, body_b, init)

            av, ai = _kv_merge_max(kx[0], ix[0],
                                   jnp.flip(kx[1]), jnp.flip(ix[1]))
            bv, bi = _kv_merge_max(kx[2], ix[2],
                                   jnp.flip(kx[3]), jnp.flip(ix[3]))
            _, itop = _kv_merge_max(av, ai, jnp.flip(bv), jnp.flip(bi))
            av, ai = _kv_merge_min(kn[0], im[0],
                                   jnp.flip(kn[1]), jnp.flip(im[1]))
            bv, bi = _kv_merge_min(kn[2], im[2],
                                   jnp.flip(kn[3]), jnp.flip(im[3]))
            _, ibot = _kv_merge_min(av, ai, jnp.flip(bv), jnp.flip(bi))

            base_t = (itop >> 4) * (G * L) + (itop & (L - 1))
            base_b = (ibot >> 4) * (G * L) + (ibot & (L - 1))
            cst = [jnp.sort(plsc.load_gather(buf, [base_t + k * L]))
                   for k in range(G)]
            tmax = _tree(cst, _merge_max)
            csb = [jnp.sort(plsc.load_gather(buf, [base_b + k * L]))
                   for k in range(G)]
            tmin = _tree(csb, _merge_min)
            out_v[pl.ds(0, K)] = tmin
            out_v[pl.ds(K, K)] = tmax
            pltpu.sync_copy(out_v, out_hbm.at[row])

        def _tree(cs, merger):
            t = cs
            while len(t) > 1:
                t = [merger(t[2 * i], t[2 * i + 1])
                     for i in range(len(t) // 2)]
            return t[0]

        bufs = (row_a, row_b)
        sems = (sem_a, sem_b)
        copies = [pltpu.async_copy(x_hbm.at[row0], row_a, sem_a)]
        for r in range(rows_per_w):
            if r + 1 < rows_per_w:
                copies.append(pltpu.async_copy(
                    x_hbm.at[row0 + r + 1], bufs[(r + 1) % 2],
                    sems[(r + 1) % 2]))
            copies[r].wait()
            compute_row(row0 + r, bufs[r % 2])

    return run(x)


# speedup vs baseline: 1.1062x; 1.1062x over previous
import dataclasses
import functools

import jax
import jax.numpy as jnp
from jax import lax
from jax.experimental import pallas as pl
from jax.experimental.pallas import tpu as pltpu
from jax.experimental.pallas import tpu_sc as plsc

ROWS = 128
COLS = 32768
K = 16
L = 16
NC = 2
NS = 16
G = 16
NG = COLS // (G * L)
U = 4


def _merge_max(a, b):
    return jnp.sort(jnp.maximum(a, jnp.flip(b)))


def _merge_min(a, b):
    return jnp.sort(jnp.minimum(a, jnp.flip(b)))


def _kv_merge_max(av, ai, bvd, bid):
    m = av >= bvd
    return plsc.sort_key_val(jnp.where(m, av, bvd), jnp.where(m, ai, bid))


def _kv_merge_min(av, ai, bvd, bid):
    m = av <= bvd
    return plsc.sort_key_val(jnp.where(m, av, bvd), jnp.where(m, ai, bid))


def kernel(x):
    nw = NC * NS
    rows_per_w = ROWS // nw

    mesh = plsc.VectorSubcoreMesh(core_axis_name="c", subcore_axis_name="s")

    cp = pltpu.CompilerParams()
    if "needs_layout_passes" in pltpu.CompilerParams.__dataclass_fields__:
        cp = dataclasses.replace(cp, needs_layout_passes=False)

    @functools.partial(
        pl.kernel,
        out_type=jax.ShapeDtypeStruct((ROWS, 2 * K), jnp.float32),
        mesh=mesh,
        compiler_params=cp,
        scratch_types=[
            pltpu.VMEM((COLS,), jnp.float32),
            pltpu.VMEM((COLS,), jnp.float32),
            pltpu.VMEM((2 * K,), jnp.float32),
            pltpu.SemaphoreType.DMA,
            pltpu.SemaphoreType.DMA,
            pltpu.SemaphoreType.DMA,
        ],
    )
    def run(x_hbm, out_hbm, row_a, row_b, out_v, sem_a, sem_b, sem_h):
        cid_ = lax.axis_index("c")
        sid_ = lax.axis_index("s")
        wid = sid_ * NC + cid_
        row0 = wid * rows_per_w

        neg = jnp.full((L,), -jnp.inf, jnp.float32)
        pos = jnp.full((L,), jnp.inf, jnp.float32)
        zero_ids = jnp.zeros((L,), jnp.int32)
        lane = lax.iota(jnp.int32, L)

        def compute_row(row, buf, glo=0, ghi=NG, carry=None):
            def body(i, carry):
                kx = list(carry[0])
                ix = list(carry[1])
                kn = list(carry[2])
                im = list(carry[3])
                for a in range(U):
                    g = i * U + a
                    gbase = g * (G * L)
                    c = buf[pl.ds(gbase, L)]
                    vmax = c
                    vmin = c
                    for k in range(1, G):
                        c = buf[pl.ds(gbase + k * L, L)]
                        vmax = jnp.maximum(vmax, c)
                        vmin = jnp.minimum(vmin, c)
                    cid = jnp.broadcast_to(g * L, (L,)).astype(jnp.int32) + lane
                    svd, sid = plsc.sort_key_val(vmax, cid, descending=True)
                    kx[a], ix[a] = _kv_merge_max(kx[a], ix[a], svd, sid)
                    svd, sid = plsc.sort_key_val(vmin, cid, descending=True)
                    kn[a], im[a] = _kv_merge_min(kn[a], im[a], svd, sid)
                return tuple(kx), tuple(ix), tuple(kn), tuple(im)

            if carry is None:
                carry = ((neg,) * U, (zero_ids,) * U,
                         (pos,) * U, (zero_ids,) * U)
            carry = lax.fori_loop(glo // U, ghi // U, body, carry)
            if ghi < NG:
                return carry
            kx, ix, kn, im = carry

            av, ai = _kv_merge_max(kx[0], ix[0],
                                   jnp.flip(kx[1]), jnp.flip(ix[1]))
            bv, bi = _kv_merge_max(kx[2], ix[2],
                                   jnp.flip(kx[3]), jnp.flip(ix[3]))
            _, itop = _kv_merge_max(av, ai, jnp.flip(bv), jnp.flip(bi))
            av, ai = _kv_merge_min(kn[0], im[0],
                                   jnp.flip(kn[1]), jnp.flip(im[1]))
            bv, bi = _kv_merge_min(kn[2], im[2],
                                   jnp.flip(kn[3]), jnp.flip(im[3]))
            _, ibot = _kv_merge_min(av, ai, jnp.flip(bv), jnp.flip(bi))

            base_t = (itop >> 4) * (G * L) + (itop & (L - 1))
            base_b = (ibot >> 4) * (G * L) + (ibot & (L - 1))
            cst = [jnp.sort(plsc.load_gather(buf, [base_t + k * L]))
                   for k in range(G)]
            tmax = _tree(cst, _merge_max)
            csb = [jnp.sort(plsc.load_gather(buf, [base_b + k * L]))
                   for k in range(G)]
            tmin = _tree(csb, _merge_min)
            out_v[pl.ds(0, K)] = tmin
            out_v[pl.ds(K, K)] = tmax
            pltpu.sync_copy(out_v, out_hbm.at[row])

        def _tree(cs, merger):
            t = cs
            while len(t) > 1:
                t = [merger(t[2 * i], t[2 * i + 1])
                     for i in range(len(t) // 2)]
            return t[0]

        bufs = (row_a, row_b)
        sems = (sem_a, sem_b)
        H = COLS // 2
        half0 = pltpu.async_copy(x_hbm.at[row0, pl.ds(0, H)],
                                 row_a.at[pl.ds(0, H)], sem_h)
        copies = [pltpu.async_copy(x_hbm.at[row0, pl.ds(H, H)],
                                   row_a.at[pl.ds(H, H)], sem_a)]
        for r in range(rows_per_w):
            if r + 1 < rows_per_w:
                copies.append(pltpu.async_copy(
                    x_hbm.at[row0 + r + 1], bufs[(r + 1) % 2],
                    sems[(r + 1) % 2]))
            if r == 0:
                half0.wait()
                carry = compute_row(row0, row_a, glo=0, ghi=NG // 2)
                copies[0].wait()
                compute_row(row0, row_a, glo=NG // 2, ghi=NG, carry=carry)
            else:
                copies[r].wait()
                compute_row(row0 + r, bufs[r % 2])

    return run(x)


# speedup vs baseline: 1.1115x; 1.0048x over previous
import dataclasses
import functools

import jax
import jax.numpy as jnp
from jax import lax
from jax.experimental import pallas as pl
from jax.experimental.pallas import tpu as pltpu
from jax.experimental.pallas import tpu_sc as plsc

ROWS = 128
COLS = 32768
K = 16
L = 16
NC = 2
NS = 16
G = 16
NG = COLS // (G * L)
U = 4


def _merge_max(a, b):
    return jnp.sort(jnp.maximum(a, jnp.flip(b)))


def _merge_min(a, b):
    return jnp.sort(jnp.minimum(a, jnp.flip(b)))


def _kv_merge_max(av, ai, bvd, bid):
    m = av >= bvd
    return plsc.sort_key_val(jnp.where(m, av, bvd), jnp.where(m, ai, bid))


def _kv_merge_min(av, ai, bvd, bid):
    m = av <= bvd
    return plsc.sort_key_val(jnp.where(m, av, bvd), jnp.where(m, ai, bid))


def kernel(x):
    nw = NC * NS
    rows_per_w = ROWS // nw

    mesh = plsc.VectorSubcoreMesh(core_axis_name="c", subcore_axis_name="s")

    cp = pltpu.CompilerParams()
    if "needs_layout_passes" in pltpu.CompilerParams.__dataclass_fields__:
        cp = dataclasses.replace(cp, needs_layout_passes=False)

    @functools.partial(
        pl.kernel,
        out_type=jax.ShapeDtypeStruct((ROWS, 2 * K), jnp.float32),
        mesh=mesh,
        compiler_params=cp,
        scratch_types=[
            pltpu.VMEM((COLS,), jnp.float32),
            pltpu.VMEM((COLS,), jnp.float32),
            pltpu.VMEM((2 * K,), jnp.float32),
            pltpu.SemaphoreType.DMA,
            pltpu.SemaphoreType.DMA,
        ],
    )
    def run(x_hbm, out_hbm, row_a, row_b, out_v, sem_a, sem_b):
        cid_ = lax.axis_index("c")
        sid_ = lax.axis_index("s")
        wid = sid_ * NC + cid_
        row0 = wid * rows_per_w

        neg = jnp.full((L,), -jnp.inf, jnp.float32)
        pos = jnp.full((L,), jnp.inf, jnp.float32)
        zero_ids = jnp.zeros((L,), jnp.int32)
        lane = lax.iota(jnp.int32, L)

        def compute_row(row, buf):
            def body(i, carry):
                kx = list(carry[0])
                ix = list(carry[1])
                kn = list(carry[2])
                im = list(carry[3])
                for a in range(U):
                    g = i * U + a
                    gbase = g * (G * L)
                    c = buf[pl.ds(gbase, L)]
                    vmax = c
                    vmin = c
                    for k in range(1, G):
                        c = buf[pl.ds(gbase + k * L, L)]
                        vmax = jnp.maximum(vmax, c)
                        vmin = jnp.minimum(vmin, c)
                    cid = jnp.broadcast_to(g * L, (L,)).astype(jnp.int32) + lane
                    svd, sid = plsc.sort_key_val(vmax, cid, descending=True)
                    kx[a], ix[a] = _kv_merge_max(kx[a], ix[a], svd, sid)
                    svd, sid = plsc.sort_key_val(vmin, cid, descending=True)
                    kn[a], im[a] = _kv_merge_min(kn[a], im[a], svd, sid)
                return tuple(kx), tuple(ix), tuple(kn), tuple(im)

            init = ((neg,) * U, (zero_ids,) * U, (pos,) * U, (zero_ids,) * U)
            kx, ix, kn, im = lax.fori_loop(0, NG // U, body, init)

            av, ai = _kv_merge_max(kx[0], ix[0],
                                   jnp.flip(kx[1]), jnp.flip(ix[1]))
            bv, bi = _kv_merge_max(kx[2], ix[2],
                                   jnp.flip(kx[3]), jnp.flip(ix[3]))
            _, itop = _kv_merge_max(av, ai, jnp.flip(bv), jnp.flip(bi))
            av, ai = _kv_merge_min(kn[0], im[0],
                                   jnp.flip(kn[1]), jnp.flip(im[1]))
            bv, bi = _kv_merge_min(kn[2], im[2],
                                   jnp.flip(kn[3]), jnp.flip(im[3]))
            _, ibot = _kv_merge_min(av, ai, jnp.flip(bv), jnp.flip(bi))

            base_t = (itop >> 4) * (G * L) + (itop & (L - 1))
            base_b = (ibot >> 4) * (G * L) + (ibot & (L - 1))
            cst = [jnp.sort(plsc.load_gather(buf, [base_t + k * L]))
                   for k in range(G)]
            tmax = _tree(cst, _merge_max)
            csb = [jnp.sort(plsc.load_gather(buf, [base_b + k * L]))
                   for k in range(G)]
            tmin = _tree(csb, _merge_min)
            out_v[pl.ds(0, K)] = tmin
            out_v[pl.ds(K, K)] = tmax
            pltpu.sync_copy(out_v, out_hbm.at[row])

        def _tree(cs, merger):
            t = cs
            while len(t) > 1:
                t = [merger(t[2 * i], t[2 * i + 1])
                     for i in range(len(t) // 2)]
            return t[0]

        bufs = (row_a, row_b)
        sems = (sem_a, sem_b)
        copies = [pltpu.async_copy(x_hbm.at[row0], row_a, sem_a)]
        for r in range(rows_per_w):
            if r + 1 < rows_per_w:
                copies.append(pltpu.async_copy(
                    x_hbm.at[row0 + r + 1], bufs[(r + 1) % 2],
                    sems[(r + 1) % 2]))
            copies[r].wait()
            compute_row(row0 + r, bufs[r % 2])

    return run(x)
